# baseline (speedup 1.0000x reference)
import functools

import jax
import jax.numpy as jnp
from jax import lax
from jax.experimental import pallas as pl
from jax.experimental.pallas import tpu as pltpu
from jax.experimental.pallas import tpu_sc as plsc

KT = 512


RT = 512


QPTS = (2736, 5472)


def _dist_argmin_body(zf_ref, emb_ref, idx_ref, rowmin_ref,
                      qmin_s, qarg_s, rmin_s):
    k = pl.program_id(1)
    nk = pl.num_programs(1)
    zb = zf_ref[...]
    eb = emb_ref[...]
    scores = lax.dot_general(zb, eb, (((1,), (1,)), ((), ())),
                             preferred_element_type=jnp.float32)
    z2 = jnp.sum(zb * zb, axis=1, keepdims=True)
    e2 = jnp.sum(eb * eb, axis=1)[None, :]
    dist = (z2 + e2) - 2.0 * scores
    cols = lax.broadcasted_iota(jnp.int32, dist.shape, 1) + k * KT
    INT_BIG = jnp.int32(2**31 - 1)
    INF = jnp.float32(jnp.inf)

    def seg_minarg(d):
        lmin = jnp.min(d, axis=1, keepdims=True)
        larg = jnp.min(jnp.where(d == lmin, cols, INT_BIG),
                       axis=1, keepdims=True)
        return lmin, larg

    def combine(lmin, larg):
        pmin = qmin_s[...]
        parg = qarg_s[...]
        upd = lmin < pmin
        qmin_s[...] = jnp.where(upd, lmin, pmin)
        qarg_s[...] = jnp.where(upd, larg, parg)

    def quantize():
        qmin_s[...] = qmin_s[...].astype(jnp.bfloat16).astype(jnp.float32)

    @pl.when(k == 0)
    def _():
        qmin_s[...] = jnp.full((dist.shape[0], 1), INF, jnp.float32)
        qarg_s[...] = jnp.full((dist.shape[0], 1), INT_BIG, jnp.int32)
        rmin_s[...] = jnp.full((dist.shape[0], 1), INF, jnp.float32)

    qs = {q // KT: q % KT for q in QPTS}
    normal = jnp.bool_(True)
    for t in qs:
        normal = jnp.logical_and(normal, k != t)

    @pl.when(normal)
    def _():
        combine(*seg_minarg(dist))

    for t, off in qs.items():
        @pl.when(k == t)
        def _(off=off):
            local = lax.broadcasted_iota(jnp.int32, dist.shape, 1)
            combine(*seg_minarg(jnp.where(local < off, dist, INF)))
            quantize()
            combine(*seg_minarg(jnp.where(local >= off, dist, INF)))

    lmin_full = jnp.min(dist, axis=1, keepdims=True)
    rmin_s[...] = jnp.minimum(rmin_s[...], lmin_full)

    @pl.when(k == nk - 1)
    def _():
        idx_ref[0] = qarg_s[...].T
        rowmin_ref[0] = rmin_s[...].T


def _dist_argmin(zf, embeddings):
    N, D = zf.shape
    K = embeddings.shape[0]
    nk = K // KT
    nr = N // RT
    return pl.pallas_call(
        _dist_argmin_body,
        grid=(nr, nk),
        in_specs=[
            pl.BlockSpec((RT, D), lambda r, k: (r, 0)),
            pl.BlockSpec((KT, D), lambda r, k: (k, 0)),
        ],
        out_specs=[
            pl.BlockSpec((1, 1, RT), lambda r, k: (r, 0, 0)),
            pl.BlockSpec((1, 1, RT), lambda r, k: (r, 0, 0)),
        ],
        out_shape=[
            jax.ShapeDtypeStruct((nr, 1, RT), jnp.int32),
            jax.ShapeDtypeStruct((nr, 1, RT), jnp.float32),
        ],
        scratch_shapes=[
            pltpu.VMEM((RT, 1), jnp.float32),
            pltpu.VMEM((RT, 1), jnp.int32),
            pltpu.VMEM((RT, 1), jnp.float32),
        ],
    )(zf, embeddings)


def _make_sc_gather(V, D, N, K):
    info = plsc.get_sparse_core_info()
    NC, NS, L = info.num_cores, info.num_subcores, info.num_lanes
    NW = NC * NS
    n_per_w = N // NW
    CH = 128
    nch = n_per_w // CH
    mesh = plsc.VectorSubcoreMesh(core_axis_name="c", subcore_axis_name="s")

    @functools.partial(
        pl.kernel, mesh=mesh,
        compiler_params=pltpu.CompilerParams(needs_layout_passes=False),
        out_type=[jax.ShapeDtypeStruct((N, D), jnp.float32),
                  jax.ShapeDtypeStruct((NW, K), jnp.float32)],
        scratch_types=[
            pltpu.VMEM((nch, CH), jnp.int32),
            pltpu.VMEM((CH, D), jnp.float32),
            pltpu.VMEM((CH, D), jnp.float32),
            pltpu.VMEM((K,), jnp.float32),
            pltpu.SemaphoreType.DMA,
            pltpu.SemaphoreType.DMA,
        ],
    )
    def sc_kernel(idx_hbm, table_hbm, out_hbm, cnt_hbm,
                  idx_v, rows0, rows1, cnt_v, sem0, sem1):
        wid = lax.axis_index("s") * NC + lax.axis_index("c")
        base = wid * n_per_w
        pltpu.sync_copy(idx_hbm.at[wid], idx_v)

        bufs = (rows0, rows1)
        sems = (sem0, sem1)
        cps = [pltpu.async_copy(table_hbm.at[idx_v.at[0]], rows0, sem0)]
        for c in range(nch):
            if c + 1 < nch:
                cps.append(pltpu.async_copy(
                    table_hbm.at[idx_v.at[c + 1]],
                    bufs[(c + 1) % 2], sems[(c + 1) % 2]))
            cps[c].wait()
            pltpu.sync_copy(bufs[c % 2],
                            out_hbm.at[pl.ds(base + c * CH, CH)])

        zeros = jnp.zeros((L,), jnp.float32)

        def _zero(i, carry):
            cnt_v[pl.ds(i * L, L)] = zeros
            return carry

        lax.fori_loop(0, K // L, _zero, 0)
        ones = jnp.ones((L,), jnp.float32)
        for c in range(nch):
            for j in range(CH // L):
                v = idx_v[c, pl.ds(j * L, L)]
                plsc.addupdate_scatter(cnt_v, [v], ones)
        pltpu.sync_copy(cnt_v, cnt_hbm.at[wid])

    return sc_kernel


def _finish_body(g_ref, cnt_ref, rowmin_ref,
                 zvq_ref, qut_ref, enc_ref, perp_ref):
    b = pl.program_id(0)
    zvq_ref[0] = g_ref[0].T

    @pl.when(b == 0)
    def _():
        n = rowmin_ref.shape[0] * rowmin_ref.shape[1]
        cnt = jnp.sum(cnt_ref[...], axis=0)
        probs = cnt * (1.0 / n)
        ent = jnp.sum(probs * jnp.log(probs + 1e-10))
        perp_ref[0, 0] = jnp.exp(-ent)
        loss = jnp.sum(rowmin_ref[...]) * (1.0 / n)
        qut_ref[0, 0] = loss
        enc_ref[0, 0] = loss


def _finish(gathered, counts, rowmin):
    B, T, D = gathered.shape
    NW, K = counts.shape
    return pl.pallas_call(
        _finish_body,
        grid=(B,),
        in_specs=[
            pl.BlockSpec((1, T, D), lambda b: (b, 0, 0)),
            pl.BlockSpec((NW, K), lambda b: (0, 0)),
            pl.BlockSpec((B, T), lambda b: (0, 0)),
        ],
        out_specs=[
            pl.BlockSpec((1, D, T), lambda b: (b, 0, 0)),
            pl.BlockSpec(memory_space=pltpu.SMEM),
            pl.BlockSpec(memory_space=pltpu.SMEM),
            pl.BlockSpec(memory_space=pltpu.SMEM),
        ],
        out_shape=[
            jax.ShapeDtypeStruct((B, D, T), jnp.float32),
            jax.ShapeDtypeStruct((1, 1), jnp.float32),
            jax.ShapeDtypeStruct((1, 1), jnp.float32),
            jax.ShapeDtypeStruct((1, 1), jnp.float32),
        ],
    )(gathered, counts, rowmin)


def kernel(z, embeddings):
    B, D, T = z.shape
    K = embeddings.shape[0]
    N = B * T

    zf = jnp.transpose(z, (0, 2, 1)).reshape(N, D)
    idx3, rowmin3 = _dist_argmin(zf, embeddings)

    info = plsc.get_sparse_core_info()
    NW = info.num_cores * info.num_subcores
    idx_tiled = idx3.reshape(NW, (N // NW) // 128, 128)
    sc = _make_sc_gather(K, D, N, K)
    gathered, counts = sc(idx_tiled, embeddings)

    zvq, qut, enc, perp = _finish(gathered.reshape(B, T, D), counts,
                                  rowmin3.reshape(B, T))
    return (zvq, qut.reshape(()), enc.reshape(()), perp.reshape(()))


# speedup vs baseline: 1.0339x; 1.0339x over previous
import functools

import jax
import jax.numpy as jnp
from jax import lax
from jax.experimental import pallas as pl
from jax.experimental.pallas import tpu as pltpu
from jax.experimental.pallas import tpu_sc as plsc

KT = 512


RT = 512


QPTS = (2736, 5472)


def _dist_argmin_body(zf_ref, emb_ref, idx_ref, rowmin_ref,
                      qmin_s, qarg_s, rmin_s, z2_s, e2_s):
    k = pl.program_id(1)
    nk = pl.num_programs(1)
    zb = zf_ref[...]
    eb = emb_ref[...]

    @pl.when(k == 0)
    def _():
        z2_s[...] = jnp.sum(zb * zb, axis=1, keepdims=True)

    r = pl.program_id(0)

    @pl.when(r == 0)
    def _():
        e2_s[0, pl.ds(k * KT, KT)] = jnp.sum(eb * eb, axis=1)

    scores = lax.dot_general(zb, eb, (((1,), (1,)), ((), ())),
                             preferred_element_type=jnp.float32)
    z2 = z2_s[...]
    e2 = e2_s[0, pl.ds(k * KT, KT)][None, :]
    dist = (z2 + e2) - 2.0 * scores
    cols = lax.broadcasted_iota(jnp.int32, dist.shape, 1)
    INT_BIG = jnp.int32(2**31 - 1)
    INF = jnp.float32(jnp.inf)

    def seg_minarg(d):
        lmin = jnp.min(d, axis=1, keepdims=True)
        larg = jnp.min(jnp.where(d == lmin, cols, INT_BIG),
                       axis=1, keepdims=True) + k * KT
        return lmin, larg

    def combine(lmin, larg):
        pmin = qmin_s[...]
        parg = qarg_s[...]
        upd = lmin < pmin
        qmin_s[...] = jnp.where(upd, lmin, pmin)
        qarg_s[...] = jnp.where(upd, larg, parg)
        rmin_s[...] = jnp.minimum(rmin_s[...], lmin)

    def quantize():
        qmin_s[...] = qmin_s[...].astype(jnp.bfloat16).astype(jnp.float32)

    @pl.when(k == 0)
    def _():
        qmin_s[...] = jnp.full((RT, 1), INF, jnp.float32)
        qarg_s[...] = jnp.full((RT, 1), INT_BIG, jnp.int32)
        rmin_s[...] = jnp.full((RT, 1), INF, jnp.float32)

    qs = {q // KT: q % KT for q in QPTS}
    normal = jnp.bool_(True)
    for t in qs:
        normal = jnp.logical_and(normal, k != t)

    @pl.when(normal)
    def _():
        combine(*seg_minarg(dist))

    for t, off in qs.items():
        @pl.when(k == t)
        def _(off=off):
            combine(*seg_minarg(jnp.where(cols < off, dist, INF)))
            quantize()
            combine(*seg_minarg(jnp.where(cols >= off, dist, INF)))

    @pl.when(k == nk - 1)
    def _():
        idx_ref[0] = qarg_s[...].T
        rowmin_ref[0] = rmin_s[...].T


def _dist_argmin(zf, embeddings):
    N, D = zf.shape
    K = embeddings.shape[0]
    nk = K // KT
    nr = N // RT
    return pl.pallas_call(
        _dist_argmin_body,
        grid=(nr, nk),
        in_specs=[
            pl.BlockSpec((RT, D), lambda r, k: (r, 0)),
            pl.BlockSpec((KT, D), lambda r, k: (k, 0)),
        ],
        out_specs=[
            pl.BlockSpec((1, 1, RT), lambda r, k: (r, 0, 0)),
            pl.BlockSpec((1, 1, RT), lambda r, k: (r, 0, 0)),
        ],
        out_shape=[
            jax.ShapeDtypeStruct((nr, 1, RT), jnp.int32),
            jax.ShapeDtypeStruct((nr, 1, RT), jnp.float32),
        ],
        scratch_shapes=[
            pltpu.VMEM((RT, 1), jnp.float32),
            pltpu.VMEM((RT, 1), jnp.int32),
            pltpu.VMEM((RT, 1), jnp.float32),
            pltpu.VMEM((RT, 1), jnp.float32),
            pltpu.VMEM((1, K), jnp.float32),
        ],
    )(zf, embeddings)


def _make_sc_gather(V, D, N, K):
    info = plsc.get_sparse_core_info()
    NC, NS, L = info.num_cores, info.num_subcores, info.num_lanes
    NW = NC * NS
    n_per_w = N // NW
    CH = 128
    nch = n_per_w // CH
    mesh = plsc.VectorSubcoreMesh(core_axis_name="c", subcore_axis_name="s")

    @functools.partial(
        pl.kernel, mesh=mesh,
        compiler_params=pltpu.CompilerParams(needs_layout_passes=False),
        out_type=[jax.ShapeDtypeStruct((N, D), jnp.float32),
                  jax.ShapeDtypeStruct((NW, K), jnp.float32)],
        scratch_types=[
            pltpu.VMEM((nch, CH), jnp.int32),
            pltpu.VMEM((CH, D), jnp.float32),
            pltpu.VMEM((CH, D), jnp.float32),
            pltpu.VMEM((K,), jnp.float32),
            pltpu.SemaphoreType.DMA,
            pltpu.SemaphoreType.DMA,
        ],
    )
    def sc_kernel(idx_hbm, table_hbm, out_hbm, cnt_hbm,
                  idx_v, rows0, rows1, cnt_v, sem0, sem1):
        wid = lax.axis_index("s") * NC + lax.axis_index("c")
        base = wid * n_per_w
        pltpu.sync_copy(idx_hbm.at[wid], idx_v)

        bufs = (rows0, rows1)
        sems = (sem0, sem1)
        cps = [pltpu.async_copy(table_hbm.at[idx_v.at[0]], rows0, sem0)]
        for c in range(nch):
            if c + 1 < nch:
                cps.append(pltpu.async_copy(
                    table_hbm.at[idx_v.at[c + 1]],
                    bufs[(c + 1) % 2], sems[(c + 1) % 2]))
            cps[c].wait()
            pltpu.sync_copy(bufs[c % 2],
                            out_hbm.at[pl.ds(base + c * CH, CH)])

        zeros = jnp.zeros((L,), jnp.float32)

        def _zero(i, carry):
            cnt_v[pl.ds(i * L, L)] = zeros
            return carry

        lax.fori_loop(0, K // L, _zero, 0)
        ones = jnp.ones((L,), jnp.float32)
        for c in range(nch):
            for j in range(CH // L):
                v = idx_v[c, pl.ds(j * L, L)]
                plsc.addupdate_scatter(cnt_v, [v], ones)
        pltpu.sync_copy(cnt_v, cnt_hbm.at[wid])

    return sc_kernel


def _finish_body(g_ref, cnt_ref, rowmin_ref,
                 zvq_ref, qut_ref, enc_ref, perp_ref):
    b = pl.program_id(0)
    zvq_ref[0] = g_ref[0].T

    @pl.when(b == 0)
    def _():
        n = rowmin_ref.shape[0] * rowmin_ref.shape[1]
        cnt = jnp.sum(cnt_ref[...], axis=0)
        probs = cnt * (1.0 / n)
        ent = jnp.sum(probs * jnp.log(probs + 1e-10))
        perp_ref[0, 0] = jnp.exp(-ent)
        loss = jnp.sum(rowmin_ref[...]) * (1.0 / n)
        qut_ref[0, 0] = loss
        enc_ref[0, 0] = loss


def _finish(gathered, counts, rowmin):
    B, T, D = gathered.shape
    NW, K = counts.shape
    return pl.pallas_call(
        _finish_body,
        grid=(B,),
        in_specs=[
            pl.BlockSpec((1, T, D), lambda b: (b, 0, 0)),
            pl.BlockSpec((NW, K), lambda b: (0, 0)),
            pl.BlockSpec((B, T), lambda b: (0, 0)),
        ],
        out_specs=[
            pl.BlockSpec((1, D, T), lambda b: (b, 0, 0)),
            pl.BlockSpec(memory_space=pltpu.SMEM),
            pl.BlockSpec(memory_space=pltpu.SMEM),
            pl.BlockSpec(memory_space=pltpu.SMEM),
        ],
        out_shape=[
            jax.ShapeDtypeStruct((B, D, T), jnp.float32),
            jax.ShapeDtypeStruct((1, 1), jnp.float32),
            jax.ShapeDtypeStruct((1, 1), jnp.float32),
            jax.ShapeDtypeStruct((1, 1), jnp.float32),
        ],
    )(gathered, counts, rowmin)


def kernel(z, embeddings):
    B, D, T = z.shape
    K = embeddings.shape[0]
    N = B * T

    zf = jnp.transpose(z, (0, 2, 1)).reshape(N, D)
    idx3, rowmin3 = _dist_argmin(zf, embeddings)

    info = plsc.get_sparse_core_info()
    NW = info.num_cores * info.num_subcores
    idx_tiled = idx3.reshape(NW, (N // NW) // 128, 128)
    sc = _make_sc_gather(K, D, N, K)
    gathered, counts = sc(idx_tiled, embeddings)

    zvq, qut, enc, perp = _finish(gathered.reshape(B, T, D), counts,
                                  rowmin3.reshape(B, T))
    return (zvq, qut.reshape(()), enc.reshape(()), perp.reshape(()))


# speedup vs baseline: 1.6891x; 1.6337x over previous
import functools

import jax
import jax.numpy as jnp
from jax import lax
from jax.experimental import pallas as pl
from jax.experimental.pallas import tpu as pltpu
from jax.experimental.pallas import tpu_sc as plsc

KT = 1024


RT = 1024


QPTS = (2736, 5472)


def _dist_argmin_body(zf_ref, emb_ref, idx_ref, rowmin_ref,
                      qmin_s, qarg_s, rmin_s, z2_s, e2_s):
    k = pl.program_id(1)
    nk = pl.num_programs(1)
    zb = zf_ref[...]
    eb = emb_ref[...]

    @pl.when(k == 0)
    def _():
        z2_s[...] = jnp.sum(zb * zb, axis=1, keepdims=True)

    r = pl.program_id(0)

    @pl.when(r == 0)
    def _():
        e2_s[0, pl.ds(k * KT, KT)] = jnp.sum(eb * eb, axis=1)

    scores = lax.dot_general(zb, eb, (((1,), (1,)), ((), ())),
                             preferred_element_type=jnp.float32)
    z2 = z2_s[...]
    e2 = e2_s[0, pl.ds(k * KT, KT)][None, :]
    dist = (z2 + e2) - 2.0 * scores
    cols = lax.broadcasted_iota(jnp.int32, dist.shape, 1)
    INT_BIG = jnp.int32(2**31 - 1)
    INF = jnp.float32(jnp.inf)

    def seg_minarg(d):
        lmin = jnp.min(d, axis=1, keepdims=True)
        larg = jnp.min(jnp.where(d == lmin, cols, INT_BIG),
                       axis=1, keepdims=True) + k * KT
        return lmin, larg

    def combine(lmin, larg):
        pmin = qmin_s[...]
        parg = qarg_s[...]
        upd = lmin < pmin
        qmin_s[...] = jnp.where(upd, lmin, pmin)
        qarg_s[...] = jnp.where(upd, larg, parg)
        rmin_s[...] = jnp.minimum(rmin_s[...], lmin)

    def quantize():
        qmin_s[...] = qmin_s[...].astype(jnp.bfloat16).astype(jnp.float32)

    @pl.when(k == 0)
    def _():
        qmin_s[...] = jnp.full((RT, 1), INF, jnp.float32)
        qarg_s[...] = jnp.full((RT, 1), INT_BIG, jnp.int32)
        rmin_s[...] = jnp.full((RT, 1), INF, jnp.float32)

    qs = {q // KT: q % KT for q in QPTS}
    normal = jnp.bool_(True)
    for t in qs:
        normal = jnp.logical_and(normal, k != t)

    @pl.when(normal)
    def _():
        combine(*seg_minarg(dist))

    for t, off in qs.items():
        @pl.when(k == t)
        def _(off=off):
            combine(*seg_minarg(jnp.where(cols < off, dist, INF)))
            quantize()
            combine(*seg_minarg(jnp.where(cols >= off, dist, INF)))

    @pl.when(k == nk - 1)
    def _():
        idx_ref[0] = qarg_s[...].T
        rowmin_ref[0] = rmin_s[...].T


def _dist_argmin(zf, embeddings):
    N, D = zf.shape
    K = embeddings.shape[0]
    nk = K // KT
    nr = N // RT
    return pl.pallas_call(
        _dist_argmin_body,
        grid=(nr, nk),
        in_specs=[
            pl.BlockSpec((RT, D), lambda r, k: (r, 0)),
            pl.BlockSpec((KT, D), lambda r, k: (k, 0)),
        ],
        out_specs=[
            pl.BlockSpec((1, 1, RT), lambda r, k: (r, 0, 0)),
            pl.BlockSpec((1, 1, RT), lambda r, k: (r, 0, 0)),
        ],
        out_shape=[
            jax.ShapeDtypeStruct((nr, 1, RT), jnp.int32),
            jax.ShapeDtypeStruct((nr, 1, RT), jnp.float32),
        ],
        scratch_shapes=[
            pltpu.VMEM((RT, 1), jnp.float32),
            pltpu.VMEM((RT, 1), jnp.int32),
            pltpu.VMEM((RT, 1), jnp.float32),
            pltpu.VMEM((RT, 1), jnp.float32),
            pltpu.VMEM((1, K), jnp.float32),
        ],
    )(zf, embeddings)


def _make_sc_gather(V, D, N, K):
    info = plsc.get_sparse_core_info()
    NC, NS, L = info.num_cores, info.num_subcores, info.num_lanes
    NW = NC * NS
    n_per_w = N // NW
    CH = 128
    nch = n_per_w // CH
    mesh = plsc.VectorSubcoreMesh(core_axis_name="c", subcore_axis_name="s")

    @functools.partial(
        pl.kernel, mesh=mesh,
        compiler_params=pltpu.CompilerParams(needs_layout_passes=False),
        out_type=[jax.ShapeDtypeStruct((N, D), jnp.float32),
                  jax.ShapeDtypeStruct((NW, K), jnp.float32)],
        scratch_types=[
            pltpu.VMEM((nch, CH), jnp.int32),
            pltpu.VMEM((CH, D), jnp.float32),
            pltpu.VMEM((CH, D), jnp.float32),
            pltpu.VMEM((K,), jnp.float32),
            pltpu.SemaphoreType.DMA,
            pltpu.SemaphoreType.DMA,
        ],
    )
    def sc_kernel(idx_hbm, table_hbm, out_hbm, cnt_hbm,
                  idx_v, rows0, rows1, cnt_v, sem0, sem1):
        wid = lax.axis_index("s") * NC + lax.axis_index("c")
        base = wid * n_per_w
        pltpu.sync_copy(idx_hbm.at[wid], idx_v)

        bufs = (rows0, rows1)
        sems = (sem0, sem1)
        cps = [pltpu.async_copy(table_hbm.at[idx_v.at[0]], rows0, sem0)]
        for c in range(nch):
            if c + 1 < nch:
                cps.append(pltpu.async_copy(
                    table_hbm.at[idx_v.at[c + 1]],
                    bufs[(c + 1) % 2], sems[(c + 1) % 2]))
            cps[c].wait()
            pltpu.sync_copy(bufs[c % 2],
                            out_hbm.at[pl.ds(base + c * CH, CH)])

        zeros = jnp.zeros((L,), jnp.float32)

        def _zero(i, carry):
            cnt_v[pl.ds(i * L, L)] = zeros
            return carry

        lax.fori_loop(0, K // L, _zero, 0)
        ones = jnp.ones((L,), jnp.float32)
        for c in range(nch):
            for j in range(CH // L):
                v = idx_v[c, pl.ds(j * L, L)]
                plsc.addupdate_scatter(cnt_v, [v], ones)
        pltpu.sync_copy(cnt_v, cnt_hbm.at[wid])

    return sc_kernel


def _finish_body(g_ref, cnt_ref, rowmin_ref,
                 zvq_ref, qut_ref, enc_ref, perp_ref):
    b = pl.program_id(0)
    zvq_ref[0] = g_ref[0].T

    @pl.when(b == 0)
    def _():
        n = rowmin_ref.shape[0] * rowmin_ref.shape[1]
        cnt = jnp.sum(cnt_ref[...], axis=0)
        probs = cnt * (1.0 / n)
        ent = jnp.sum(probs * jnp.log(probs + 1e-10))
        perp_ref[0, 0] = jnp.exp(-ent)
        loss = jnp.sum(rowmin_ref[...]) * (1.0 / n)
        qut_ref[0, 0] = loss
        enc_ref[0, 0] = loss


def _finish(gathered, counts, rowmin):
    B, T, D = gathered.shape
    NW, K = counts.shape
    return pl.pallas_call(
        _finish_body,
        grid=(B,),
        in_specs=[
            pl.BlockSpec((1, T, D), lambda b: (b, 0, 0)),
            pl.BlockSpec((NW, K), lambda b: (0, 0)),
            pl.BlockSpec((B, T), lambda b: (0, 0)),
        ],
        out_specs=[
            pl.BlockSpec((1, D, T), lambda b: (b, 0, 0)),
            pl.BlockSpec(memory_space=pltpu.SMEM),
            pl.BlockSpec(memory_space=pltpu.SMEM),
            pl.BlockSpec(memory_space=pltpu.SMEM),
        ],
        out_shape=[
            jax.ShapeDtypeStruct((B, D, T), jnp.float32),
            jax.ShapeDtypeStruct((1, 1), jnp.float32),
            jax.ShapeDtypeStruct((1, 1), jnp.float32),
            jax.ShapeDtypeStruct((1, 1), jnp.float32),
        ],
    )(gathered, counts, rowmin)


def kernel(z, embeddings):
    B, D, T = z.shape
    K = embeddings.shape[0]
    N = B * T

    zf = jnp.transpose(z, (0, 2, 1)).reshape(N, D)
    idx3, rowmin3 = _dist_argmin(zf, embeddings)

    info = plsc.get_sparse_core_info()
    NW = info.num_cores * info.num_subcores
    idx_tiled = idx3.reshape(NW, (N // NW) // 128, 128)
    sc = _make_sc_gather(K, D, N, K)
    gathered, counts = sc(idx_tiled, embeddings)

    zvq, qut, enc, perp = _finish(gathered.reshape(B, T, D), counts,
                                  rowmin3.reshape(B, T))
    return (zvq, qut.reshape(()), enc.reshape(()), perp.reshape(()))
